# Initial kernel scaffold; baseline (speedup 1.0000x reference)
import jax
import jax.numpy as jnp
from jax.experimental import pallas as pl


def kernel(x, edge_index, edge_attr, params):
    raise NotImplementedError("write your pallas kernel here")


# speedup vs baseline: 2.7407x; 2.7407x over previous
import functools

import jax
import jax.numpy as jnp
from jax import lax
from jax.experimental import pallas as pl
from jax.experimental.pallas import tpu as pltpu
from jax.experimental.pallas import tpu_sc as plsc

N_NODES = 10000
N_EDGES = 160000
D_NODE = 32
E_BLK = 1280
CHUNK = 128
N_CHUNKS = N_EDGES // CHUNK
NC = 2
NS = 16
NW = NC * NS
ROWS_PER_SUB = 624
ROWS_REM = N_NODES - NS * ROWS_PER_SUB


def _gelu(v):
    return 0.5 * v * (1.0 + lax.erf(v * (2.0 ** -0.5)))


def _sc_gather_body(h_hbm, src_hbm, out_hbm, idx_v, rows_v, sem):
    cid = lax.axis_index("c")
    sid = lax.axis_index("s")
    wid = sid * NC + cid
    n_iter = (N_CHUNKS + NW - 1) // NW

    def body(j, _):
        c = j * NW + wid

        @pl.when(c < N_CHUNKS)
        def _():
            pltpu.sync_copy(src_hbm.at[pl.ds(c * CHUNK, CHUNK)], idx_v)
            pltpu.async_copy(h_hbm.at[idx_v], rows_v, sem).wait()
            pltpu.sync_copy(rows_v, out_hbm.at[pl.ds(c * CHUNK, CHUNK)])

        return 0

    lax.fori_loop(0, n_iter, body, 0)


def _sc_gather(h, src):
    mesh = plsc.VectorSubcoreMesh(core_axis_name="c", subcore_axis_name="s")
    return pl.kernel(
        _sc_gather_body,
        out_type=jax.ShapeDtypeStruct((N_EDGES, D_NODE), jnp.float32),
        mesh=mesh,
        compiler_params=pltpu.CompilerParams(use_tc_tiling_on_sc=False),
        scratch_types=[
            pltpu.VMEM((CHUNK,), jnp.int32),
            pltpu.VMEM((CHUNK, D_NODE), jnp.float32),
            pltpu.SemaphoreType.DMA,
        ],
    )(h, src)


def _sc_scatter_body(width, msg_hbm, dst_hbm, zeros_hbm, out_hbm,
                     idx_v, rows_v, acc, sem):
    cid = lax.axis_index("c")
    sid = lax.axis_index("s")
    pltpu.sync_copy(zeros_hbm.at[pl.ds(sid * ROWS_PER_SUB, ROWS_PER_SUB)],
                    acc.at[pl.ds(sid * ROWS_PER_SUB, ROWS_PER_SUB)])

    @pl.when(sid == 0)
    def _():
        pltpu.sync_copy(zeros_hbm.at[pl.ds(NS * ROWS_PER_SUB, ROWS_REM)],
                        acc.at[pl.ds(NS * ROWS_PER_SUB, ROWS_REM)])

    plsc.subcore_barrier()

    per_core = N_CHUNKS // NC
    n_iter = (per_core + NS - 1) // NS

    def body(j, _):
        cl = j * NS + sid

        @pl.when(cl < per_core)
        def _():
            c = cid * per_core + cl
            pltpu.sync_copy(dst_hbm.at[pl.ds(c * CHUNK, CHUNK)], idx_v)
            pltpu.sync_copy(msg_hbm.at[pl.ds(c * CHUNK, CHUNK)], rows_v)
            pltpu.sync_copy(rows_v, acc.at[idx_v], add=True)

        return 0

    lax.fori_loop(0, n_iter, body, 0)
    plsc.subcore_barrier()
    pltpu.sync_copy(acc.at[pl.ds(sid * ROWS_PER_SUB, ROWS_PER_SUB)],
                    out_hbm.at[cid, pl.ds(sid * ROWS_PER_SUB, ROWS_PER_SUB)])

    @pl.when(sid == 0)
    def _():
        pltpu.sync_copy(acc.at[pl.ds(NS * ROWS_PER_SUB, ROWS_REM)],
                        out_hbm.at[cid, pl.ds(NS * ROWS_PER_SUB, ROWS_REM)])


def _sc_scatter_add(msg, dst, width):
    mesh = plsc.VectorSubcoreMesh(core_axis_name="c", subcore_axis_name="s")
    zeros = jnp.zeros((N_NODES, width), jnp.float32)
    return pl.kernel(
        functools.partial(_sc_scatter_body, width),
        out_type=jax.ShapeDtypeStruct((NC, N_NODES, width), jnp.float32),
        mesh=mesh,
        compiler_params=pltpu.CompilerParams(use_tc_tiling_on_sc=False),
        scratch_types=[
            pltpu.VMEM((CHUNK,), jnp.int32),
            pltpu.VMEM((CHUNK, width), jnp.float32),
            pltpu.VMEM_SHARED((N_NODES, width), jnp.float32),
            pltpu.SemaphoreType.DMA,
        ],
    )(msg, dst, zeros)


def _fc1_body(x_ref, w_ref, b_ref, cnt_ref, h_ref, inv_ref):
    h_ref[...] = jnp.dot(x_ref[...], w_ref[...],
                         preferred_element_type=jnp.float32) + b_ref[...]
    cnt = cnt_ref[0, :, 0:1] + cnt_ref[1, :, 0:1]
    inv_ref[...] = 1.0 / jnp.maximum(cnt, 1.0)


def _fc1(x, w, b, cnt2):
    return pl.pallas_call(
        _fc1_body,
        out_shape=(
            jax.ShapeDtypeStruct((N_NODES, D_NODE), jnp.float32),
            jax.ShapeDtypeStruct((N_NODES, 1), jnp.float32),
        ),
    )(x, w, b.reshape(1, -1), cnt2)


def _edge_body(eaT_ref, hsT_ref, w1_ref, b1_ref, w2_ref, b2_ref,
               w3_ref, b3_ref, out_ref):
    e1 = jnp.maximum(
        jnp.dot(w1_ref[...], eaT_ref[...],
                preferred_element_type=jnp.float32) + b1_ref[...], 0.0)
    e2 = jnp.maximum(
        jnp.dot(w2_ref[...], e1,
                preferred_element_type=jnp.float32) + b2_ref[...], 0.0)
    wt = jnp.dot(w3_ref[...], e2,
                 preferred_element_type=jnp.float32) + b3_ref[...]
    wt3 = wt.reshape(D_NODE, D_NODE, E_BLK)
    hs = hsT_ref[...]
    out_ref[...] = jnp.sum(wt3 * hs[:, None, :], axis=0)


def _edge_msgT(eaT, hsrcT, kparams):
    (w1, b1), (w2, b2), (w3, b3) = kparams
    grid = N_EDGES // E_BLK
    full = lambda shape: pl.BlockSpec(shape, lambda j: (0, 0))
    return pl.pallas_call(
        _edge_body,
        grid=(grid,),
        in_specs=[
            pl.BlockSpec((16, E_BLK), lambda j: (0, j)),
            pl.BlockSpec((D_NODE, E_BLK), lambda j: (0, j)),
            full((64, 16)), full((64, 1)),
            full((128, 64)), full((128, 1)),
            full((1024, 128)), full((1024, 1)),
        ],
        out_specs=pl.BlockSpec((D_NODE, E_BLK), lambda j: (0, j)),
        out_shape=jax.ShapeDtypeStruct((D_NODE, N_EDGES), jnp.float32),
    )(eaT, hsrcT, w1.T, b1.reshape(-1, 1), w2.T, b2.reshape(-1, 1),
      w3.T, b3.reshape(-1, 1))


def _update_body(do_gelu, s2_ref, inv_ref, h_ref, root_ref, bias_ref, out_ref):
    aggr = (s2_ref[0] + s2_ref[1]) * inv_ref[...]
    v = aggr + jnp.dot(h_ref[...], root_ref[...],
                       preferred_element_type=jnp.float32) + bias_ref[...]
    if do_gelu:
        v = _gelu(v)
    out_ref[...] = v


def _update(s2, inv, h, root, bias, do_gelu):
    return pl.pallas_call(
        functools.partial(_update_body, do_gelu),
        out_shape=jax.ShapeDtypeStruct((N_NODES, D_NODE), jnp.float32),
    )(s2, inv, h, root, bias.reshape(1, -1))


def _head_body(h_ref, w2_ref, b2_ref, w3_ref, b3_ref, out_ref):
    v = jnp.dot(h_ref[...], w2_ref[...],
                preferred_element_type=jnp.float32) + b2_ref[...]
    v = _gelu(v)
    out_ref[...] = jnp.dot(v, w3_ref[...],
                           preferred_element_type=jnp.float32) + b3_ref[...]


def _head(h, fc2, fc3):
    return pl.pallas_call(
        _head_body,
        out_shape=jax.ShapeDtypeStruct((N_NODES, 1), jnp.float32),
    )(h, fc2[0], fc2[1].reshape(1, -1), fc3[0], fc3[1].reshape(1, -1))


def kernel(x, edge_index, edge_attr, params):
    src = edge_index[0]
    dst = edge_index[1]
    eaT = edge_attr.T

    ones_rows = jnp.ones((N_EDGES, 16), jnp.float32)
    cnt2 = _sc_scatter_add(ones_rows, dst, 16)
    h, inv = _fc1(x, params['fc1'][0], params['fc1'][1], cnt2)

    for i, c in enumerate(params['convs']):
        hsrc = _sc_gather(h, src)
        msgT = _edge_msgT(eaT, hsrc.T, c['kernel'])
        s2 = _sc_scatter_add(msgT.T, dst, D_NODE)
        h = _update(s2, inv, h, c['root'], c['bias'], do_gelu=(i < 3))

    return _head(h, params['fc2'], params['fc3'])


# speedup vs baseline: 3.3739x; 1.2311x over previous
import functools

import jax
import jax.numpy as jnp
from jax import lax
from jax.experimental import pallas as pl
from jax.experimental.pallas import tpu as pltpu
from jax.experimental.pallas import tpu_sc as plsc

N_NODES = 10000
N_EDGES = 160000
D_NODE = 32
E_BLK = 1280
CHUNK = 1000
N_CHUNKS = N_EDGES // CHUNK
NC = 2
NS = 16
NW = NC * NS
ROWS_PER_SUB = 624
ROWS_REM = N_NODES - NS * ROWS_PER_SUB


def _gelu(v):
    return 0.5 * v * (1.0 + lax.erf(v * (2.0 ** -0.5)))


def _sc_gather_body(h_hbm, src_hbm, out_hbm,
                    idx0, idx1, rows0, rows1, rows2,
                    g0, g1, o0, o1, o2):
    cid = lax.axis_index("c")
    sid = lax.axis_index("s")
    wid = sid * NC + cid
    per_w = N_CHUNKS // NW
    base = wid * per_w
    idx = [idx0, idx1]
    rows = [rows0, rows1, rows2]
    gsem = [g0, g1]
    osem = [o0, o1, o2]

    def load_idx(j):
        pltpu.sync_copy(src_hbm.at[pl.ds((base + j) * CHUNK, CHUNK)],
                        idx[j % 2])

    def fire_gather(j):
        pltpu.async_copy(h_hbm.at[idx[j % 2]], rows[j % 3], gsem[j % 2])

    load_idx(0)
    fire_gather(0)
    load_idx(1)
    fire_gather(1)
    for j in range(per_w):
        pltpu.make_async_copy(h_hbm.at[idx[j % 2]], rows[j % 3],
                              gsem[j % 2]).wait()
        pltpu.async_copy(rows[j % 3],
                         out_hbm.at[pl.ds((base + j) * CHUNK, CHUNK)],
                         osem[j % 3])
        nxt = j + 2
        if nxt < per_w:
            load_idx(nxt)
            if nxt >= 3:
                pltpu.make_async_copy(
                    rows[nxt % 3],
                    out_hbm.at[pl.ds((base + nxt - 3) * CHUNK, CHUNK)],
                    osem[nxt % 3]).wait()
            fire_gather(nxt)
    for j in range(max(0, per_w - 3), per_w):
        pltpu.make_async_copy(rows[j % 3],
                              out_hbm.at[pl.ds((base + j) * CHUNK, CHUNK)],
                              osem[j % 3]).wait()


def _sc_gather(h, src):
    mesh = plsc.VectorSubcoreMesh(core_axis_name="c", subcore_axis_name="s")
    return pl.kernel(
        _sc_gather_body,
        out_type=jax.ShapeDtypeStruct((N_EDGES, D_NODE), jnp.float32),
        mesh=mesh,
        compiler_params=pltpu.CompilerParams(use_tc_tiling_on_sc=False),
        scratch_types=[
            pltpu.VMEM((CHUNK,), jnp.int32),
            pltpu.VMEM((CHUNK,), jnp.int32),
            pltpu.VMEM((CHUNK, D_NODE), jnp.float32),
            pltpu.VMEM((CHUNK, D_NODE), jnp.float32),
            pltpu.VMEM((CHUNK, D_NODE), jnp.float32),
            pltpu.SemaphoreType.DMA,
            pltpu.SemaphoreType.DMA,
            pltpu.SemaphoreType.DMA,
            pltpu.SemaphoreType.DMA,
            pltpu.SemaphoreType.DMA,
        ],
    )(h, src)


def _sc_scatter_body(width, msg_hbm, dst_hbm, zeros_hbm, out_hbm,
                     idx0, idx1, rows0, rows1, acc,
                     i0, i1, m0, m1, ssem):
    cid = lax.axis_index("c")
    sid = lax.axis_index("s")
    pltpu.sync_copy(zeros_hbm.at[pl.ds(sid * ROWS_PER_SUB, ROWS_PER_SUB)],
                    acc.at[pl.ds(sid * ROWS_PER_SUB, ROWS_PER_SUB)])

    @pl.when(sid == 0)
    def _():
        pltpu.sync_copy(zeros_hbm.at[pl.ds(NS * ROWS_PER_SUB, ROWS_REM)],
                        acc.at[pl.ds(NS * ROWS_PER_SUB, ROWS_REM)])

    plsc.subcore_barrier()

    per_core = N_CHUNKS // NC
    per_w = per_core // NS
    base = cid * per_core + sid * per_w
    idx = [idx0, idx1]
    rows = [rows0, rows1]
    isem = [i0, i1]
    msem = [m0, m1]

    def load(j):
        b = j % 2
        pltpu.async_copy(dst_hbm.at[pl.ds((base + j) * CHUNK, CHUNK)],
                         idx[b], isem[b])
        pltpu.async_copy(msg_hbm.at[pl.ds((base + j) * CHUNK, CHUNK)],
                         rows[b], msem[b])

    load(0)
    for j in range(per_w):
        b = j % 2
        pltpu.make_async_copy(dst_hbm.at[pl.ds((base + j) * CHUNK, CHUNK)],
                              idx[b], isem[b]).wait()
        pltpu.make_async_copy(msg_hbm.at[pl.ds((base + j) * CHUNK, CHUNK)],
                              rows[b], msem[b]).wait()
        if j + 1 < per_w:
            load(j + 1)
        pltpu.async_copy(rows[b], acc.at[idx[b]], ssem, add=True)
        pltpu.make_async_copy(rows[b], acc.at[idx[b]], ssem).wait()
    plsc.subcore_barrier()
    pltpu.sync_copy(acc.at[pl.ds(sid * ROWS_PER_SUB, ROWS_PER_SUB)],
                    out_hbm.at[cid, pl.ds(sid * ROWS_PER_SUB, ROWS_PER_SUB)])

    @pl.when(sid == 0)
    def _():
        pltpu.sync_copy(acc.at[pl.ds(NS * ROWS_PER_SUB, ROWS_REM)],
                        out_hbm.at[cid, pl.ds(NS * ROWS_PER_SUB, ROWS_REM)])


def _sc_scatter_add(msg, dst, width):
    mesh = plsc.VectorSubcoreMesh(core_axis_name="c", subcore_axis_name="s")
    zeros = jnp.zeros((N_NODES, width), jnp.float32)
    return pl.kernel(
        functools.partial(_sc_scatter_body, width),
        out_type=jax.ShapeDtypeStruct((NC, N_NODES, width), jnp.float32),
        mesh=mesh,
        compiler_params=pltpu.CompilerParams(use_tc_tiling_on_sc=False),
        scratch_types=[
            pltpu.VMEM((CHUNK,), jnp.int32),
            pltpu.VMEM((CHUNK,), jnp.int32),
            pltpu.VMEM((CHUNK, width), jnp.float32),
            pltpu.VMEM((CHUNK, width), jnp.float32),
            pltpu.VMEM_SHARED((N_NODES, width), jnp.float32),
            pltpu.SemaphoreType.DMA,
            pltpu.SemaphoreType.DMA,
            pltpu.SemaphoreType.DMA,
            pltpu.SemaphoreType.DMA,
            pltpu.SemaphoreType.DMA,
        ],
    )(msg, dst, zeros)


def _fc1_body(x_ref, w_ref, b_ref, cnt_ref, h_ref, inv_ref):
    h_ref[...] = jnp.dot(x_ref[...], w_ref[...],
                         preferred_element_type=jnp.float32) + b_ref[...]
    cnt = cnt_ref[0, :, 0:1] + cnt_ref[1, :, 0:1]
    inv_ref[...] = 1.0 / jnp.maximum(cnt, 1.0)


def _fc1(x, w, b, cnt2):
    return pl.pallas_call(
        _fc1_body,
        out_shape=(
            jax.ShapeDtypeStruct((N_NODES, D_NODE), jnp.float32),
            jax.ShapeDtypeStruct((N_NODES, 1), jnp.float32),
        ),
    )(x, w, b.reshape(1, -1), cnt2)


def _edge_body(eaT_ref, hsT_ref, w1_ref, b1_ref, w2_ref, b2_ref,
               w3_ref, b3_ref, out_ref):
    e1 = jnp.maximum(
        jnp.dot(w1_ref[...], eaT_ref[...],
                preferred_element_type=jnp.float32) + b1_ref[...], 0.0)
    e2 = jnp.maximum(
        jnp.dot(w2_ref[...], e1,
                preferred_element_type=jnp.float32) + b2_ref[...], 0.0)
    wt = jnp.dot(w3_ref[...], e2,
                 preferred_element_type=jnp.float32) + b3_ref[...]
    wt3 = wt.reshape(D_NODE, D_NODE, E_BLK)
    hs = hsT_ref[...]
    out_ref[...] = jnp.sum(wt3 * hs[:, None, :], axis=0)


def _edge_msgT(eaT, hsrcT, kparams):
    (w1, b1), (w2, b2), (w3, b3) = kparams
    grid = N_EDGES // E_BLK
    full = lambda shape: pl.BlockSpec(shape, lambda j: (0, 0))
    return pl.pallas_call(
        _edge_body,
        grid=(grid,),
        in_specs=[
            pl.BlockSpec((16, E_BLK), lambda j: (0, j)),
            pl.BlockSpec((D_NODE, E_BLK), lambda j: (0, j)),
            full((64, 16)), full((64, 1)),
            full((128, 64)), full((128, 1)),
            full((1024, 128)), full((1024, 1)),
        ],
        out_specs=pl.BlockSpec((D_NODE, E_BLK), lambda j: (0, j)),
        out_shape=jax.ShapeDtypeStruct((D_NODE, N_EDGES), jnp.float32),
    )(eaT, hsrcT, w1.T, b1.reshape(-1, 1), w2.T, b2.reshape(-1, 1),
      w3.T, b3.reshape(-1, 1))


def _update_body(do_gelu, s2_ref, inv_ref, h_ref, root_ref, bias_ref, out_ref):
    aggr = (s2_ref[0] + s2_ref[1]) * inv_ref[...]
    v = aggr + jnp.dot(h_ref[...], root_ref[...],
                       preferred_element_type=jnp.float32) + bias_ref[...]
    if do_gelu:
        v = _gelu(v)
    out_ref[...] = v


def _update(s2, inv, h, root, bias, do_gelu):
    return pl.pallas_call(
        functools.partial(_update_body, do_gelu),
        out_shape=jax.ShapeDtypeStruct((N_NODES, D_NODE), jnp.float32),
    )(s2, inv, h, root, bias.reshape(1, -1))


def _head_body(h_ref, w2_ref, b2_ref, w3_ref, b3_ref, out_ref):
    v = jnp.dot(h_ref[...], w2_ref[...],
                preferred_element_type=jnp.float32) + b2_ref[...]
    v = _gelu(v)
    out_ref[...] = jnp.dot(v, w3_ref[...],
                           preferred_element_type=jnp.float32) + b3_ref[...]


def _head(h, fc2, fc3):
    return pl.pallas_call(
        _head_body,
        out_shape=jax.ShapeDtypeStruct((N_NODES, 1), jnp.float32),
    )(h, fc2[0], fc2[1].reshape(1, -1), fc3[0], fc3[1].reshape(1, -1))


def kernel(x, edge_index, edge_attr, params):
    src = edge_index[0]
    dst = edge_index[1]
    eaT = edge_attr.T

    ones_rows = jnp.ones((N_EDGES, 16), jnp.float32)
    cnt2 = _sc_scatter_add(ones_rows, dst, 16)
    h, inv = _fc1(x, params['fc1'][0], params['fc1'][1], cnt2)

    for i, c in enumerate(params['convs']):
        hsrc = _sc_gather(h, src)
        msgT = _edge_msgT(eaT, hsrc.T, c['kernel'])
        s2 = _sc_scatter_add(msgT.T, dst, D_NODE)
        h = _update(s2, inv, h, c['root'], c['bias'], do_gelu=(i < 3))

    return _head(h, params['fc2'], params['fc3'])


# speedup vs baseline: 3.6115x; 1.0704x over previous
import functools

import jax
import jax.numpy as jnp
from jax import lax
from jax.experimental import pallas as pl
from jax.experimental.pallas import tpu as pltpu
from jax.experimental.pallas import tpu_sc as plsc

N_NODES = 10000
N_EDGES = 160000
D_NODE = 32
E_BLK = 1280
CHUNK = 1000
N_CHUNKS = N_EDGES // CHUNK
NC = 2
NS = 16
NW = NC * NS
ROWS_PER_SUB = 624
ROWS_REM = N_NODES - NS * ROWS_PER_SUB


def _gelu(v):
    return 0.5 * v * (1.0 + lax.erf(v * (2.0 ** -0.5)))


def _sc_gather_body(h_hbm, src_hbm, out_hbm,
                    idx0, idx1, rows0, rows1, rows2,
                    g0, g1, o0, o1, o2):
    cid = lax.axis_index("c")
    sid = lax.axis_index("s")
    wid = sid * NC + cid
    per_w = N_CHUNKS // NW
    base = wid * per_w
    idx = [idx0, idx1]
    rows = [rows0, rows1, rows2]
    gsem = [g0, g1]
    osem = [o0, o1, o2]

    def load_idx(j):
        pltpu.sync_copy(src_hbm.at[pl.ds((base + j) * CHUNK, CHUNK)],
                        idx[j % 2])

    def fire_gather(j):
        pltpu.async_copy(h_hbm.at[idx[j % 2]], rows[j % 3], gsem[j % 2])

    load_idx(0)
    fire_gather(0)
    load_idx(1)
    fire_gather(1)
    for j in range(per_w):
        pltpu.make_async_copy(h_hbm.at[idx[j % 2]], rows[j % 3],
                              gsem[j % 2]).wait()
        pltpu.async_copy(rows[j % 3],
                         out_hbm.at[pl.ds((base + j) * CHUNK, CHUNK)],
                         osem[j % 3])
        nxt = j + 2
        if nxt < per_w:
            load_idx(nxt)
            if nxt >= 3:
                pltpu.make_async_copy(
                    rows[nxt % 3],
                    out_hbm.at[pl.ds((base + nxt - 3) * CHUNK, CHUNK)],
                    osem[nxt % 3]).wait()
            fire_gather(nxt)
    for j in range(max(0, per_w - 3), per_w):
        pltpu.make_async_copy(rows[j % 3],
                              out_hbm.at[pl.ds((base + j) * CHUNK, CHUNK)],
                              osem[j % 3]).wait()


def _sc_gather(h, src):
    mesh = plsc.VectorSubcoreMesh(core_axis_name="c", subcore_axis_name="s")
    return pl.kernel(
        _sc_gather_body,
        out_type=jax.ShapeDtypeStruct((N_EDGES, D_NODE), jnp.float32),
        mesh=mesh,
        compiler_params=pltpu.CompilerParams(use_tc_tiling_on_sc=False),
        scratch_types=[
            pltpu.VMEM((CHUNK,), jnp.int32),
            pltpu.VMEM((CHUNK,), jnp.int32),
            pltpu.VMEM((CHUNK, D_NODE), jnp.float32),
            pltpu.VMEM((CHUNK, D_NODE), jnp.float32),
            pltpu.VMEM((CHUNK, D_NODE), jnp.float32),
            pltpu.SemaphoreType.DMA,
            pltpu.SemaphoreType.DMA,
            pltpu.SemaphoreType.DMA,
            pltpu.SemaphoreType.DMA,
            pltpu.SemaphoreType.DMA,
        ],
    )(h, src)


def _sc_scatter_body(width, msg_hbm, dst_hbm, zeros_hbm, out_hbm,
                     idx0, idx1, rows0, rows1, acc,
                     i0, i1, m0, m1, ssem):
    cid = lax.axis_index("c")
    sid = lax.axis_index("s")
    pltpu.sync_copy(zeros_hbm.at[pl.ds(sid * ROWS_PER_SUB, ROWS_PER_SUB)],
                    acc.at[pl.ds(sid * ROWS_PER_SUB, ROWS_PER_SUB)])

    @pl.when(sid == 0)
    def _():
        pltpu.sync_copy(zeros_hbm.at[pl.ds(NS * ROWS_PER_SUB, ROWS_REM)],
                        acc.at[pl.ds(NS * ROWS_PER_SUB, ROWS_REM)])

    plsc.subcore_barrier()

    per_core = N_CHUNKS // NC
    per_w = per_core // NS
    base = cid * per_core + sid * per_w
    idx = [idx0, idx1]
    rows = [rows0, rows1]
    isem = [i0, i1]
    msem = [m0, m1]

    def load(j):
        b = j % 2
        pltpu.async_copy(dst_hbm.at[pl.ds((base + j) * CHUNK, CHUNK)],
                         idx[b], isem[b])
        pltpu.async_copy(msg_hbm.at[pl.ds((base + j) * CHUNK, CHUNK)],
                         rows[b], msem[b])

    load(0)
    for j in range(per_w):
        b = j % 2
        pltpu.make_async_copy(dst_hbm.at[pl.ds((base + j) * CHUNK, CHUNK)],
                              idx[b], isem[b]).wait()
        pltpu.make_async_copy(msg_hbm.at[pl.ds((base + j) * CHUNK, CHUNK)],
                              rows[b], msem[b]).wait()
        if j + 1 < per_w:
            load(j + 1)
        pltpu.async_copy(rows[b], acc.at[idx[b]], ssem, add=True)
        pltpu.make_async_copy(rows[b], acc.at[idx[b]], ssem).wait()
    plsc.subcore_barrier()
    pltpu.sync_copy(acc.at[pl.ds(sid * ROWS_PER_SUB, ROWS_PER_SUB)],
                    out_hbm.at[cid, pl.ds(sid * ROWS_PER_SUB, ROWS_PER_SUB)])

    @pl.when(sid == 0)
    def _():
        pltpu.sync_copy(acc.at[pl.ds(NS * ROWS_PER_SUB, ROWS_REM)],
                        out_hbm.at[cid, pl.ds(NS * ROWS_PER_SUB, ROWS_REM)])


def _sc_scatter_add(msg, dst, width):
    mesh = plsc.VectorSubcoreMesh(core_axis_name="c", subcore_axis_name="s")
    zeros = jnp.zeros((N_NODES, width), jnp.float32)
    return pl.kernel(
        functools.partial(_sc_scatter_body, width),
        out_type=jax.ShapeDtypeStruct((NC, N_NODES, width), jnp.float32),
        mesh=mesh,
        compiler_params=pltpu.CompilerParams(use_tc_tiling_on_sc=False),
        scratch_types=[
            pltpu.VMEM((CHUNK,), jnp.int32),
            pltpu.VMEM((CHUNK,), jnp.int32),
            pltpu.VMEM((CHUNK, width), jnp.float32),
            pltpu.VMEM((CHUNK, width), jnp.float32),
            pltpu.VMEM_SHARED((N_NODES, width), jnp.float32),
            pltpu.SemaphoreType.DMA,
            pltpu.SemaphoreType.DMA,
            pltpu.SemaphoreType.DMA,
            pltpu.SemaphoreType.DMA,
            pltpu.SemaphoreType.DMA,
        ],
    )(msg, dst, zeros)


def _fc1_body(x_ref, w_ref, b_ref, cnt_ref, h_ref, inv_ref):
    h_ref[...] = jnp.dot(x_ref[...], w_ref[...],
                         preferred_element_type=jnp.float32) + b_ref[...]
    cnt = cnt_ref[0, :, 0:1] + cnt_ref[1, :, 0:1]
    inv_ref[...] = 1.0 / jnp.maximum(cnt, 1.0)


def _fc1(x, w, b, cnt2):
    return pl.pallas_call(
        _fc1_body,
        out_shape=(
            jax.ShapeDtypeStruct((N_NODES, D_NODE), jnp.float32),
            jax.ShapeDtypeStruct((N_NODES, 1), jnp.float32),
        ),
    )(x, w, b.reshape(1, -1), cnt2)


def _edge_body(ea_ref, hs_ref, w1_ref, b1_ref, w2_ref, b2_ref,
               w3_ref, b3_ref, out_ref):
    eaT = ea_ref[...].T
    e1 = jnp.maximum(
        jnp.dot(w1_ref[...], eaT,
                preferred_element_type=jnp.float32) + b1_ref[...], 0.0)
    e2 = jnp.maximum(
        jnp.dot(w2_ref[...], e1,
                preferred_element_type=jnp.float32) + b2_ref[...], 0.0)
    wt = jnp.dot(w3_ref[...], e2,
                 preferred_element_type=jnp.float32) + b3_ref[...]
    wt3 = wt.reshape(D_NODE, D_NODE, E_BLK)
    hs = hs_ref[...].T
    msgT = jnp.sum(wt3 * hs[:, None, :], axis=0)
    out_ref[...] = msgT.T


def _edge_msg(ea, hsrc, kparams):
    (w1, b1), (w2, b2), (w3, b3) = kparams
    grid = N_EDGES // E_BLK
    full = lambda shape: pl.BlockSpec(shape, lambda j: (0, 0))
    return pl.pallas_call(
        _edge_body,
        grid=(grid,),
        in_specs=[
            pl.BlockSpec((E_BLK, 16), lambda j: (j, 0)),
            pl.BlockSpec((E_BLK, D_NODE), lambda j: (j, 0)),
            full((64, 16)), full((64, 1)),
            full((128, 64)), full((128, 1)),
            full((1024, 128)), full((1024, 1)),
        ],
        out_specs=pl.BlockSpec((E_BLK, D_NODE), lambda j: (j, 0)),
        out_shape=jax.ShapeDtypeStruct((N_EDGES, D_NODE), jnp.float32),
    )(ea, hsrc, w1.T, b1.reshape(-1, 1), w2.T, b2.reshape(-1, 1),
      w3.T, b3.reshape(-1, 1))


def _update_body(do_gelu, s2_ref, inv_ref, h_ref, root_ref, bias_ref, out_ref):
    aggr = (s2_ref[0] + s2_ref[1]) * inv_ref[...]
    v = aggr + jnp.dot(h_ref[...], root_ref[...],
                       preferred_element_type=jnp.float32) + bias_ref[...]
    if do_gelu:
        v = _gelu(v)
    out_ref[...] = v


def _update(s2, inv, h, root, bias, do_gelu):
    return pl.pallas_call(
        functools.partial(_update_body, do_gelu),
        out_shape=jax.ShapeDtypeStruct((N_NODES, D_NODE), jnp.float32),
    )(s2, inv, h, root, bias.reshape(1, -1))


def _head_body(h_ref, w2_ref, b2_ref, w3_ref, b3_ref, out_ref):
    v = jnp.dot(h_ref[...], w2_ref[...],
                preferred_element_type=jnp.float32) + b2_ref[...]
    v = _gelu(v)
    out_ref[...] = jnp.dot(v, w3_ref[...],
                           preferred_element_type=jnp.float32) + b3_ref[...]


def _head(h, fc2, fc3):
    return pl.pallas_call(
        _head_body,
        out_shape=jax.ShapeDtypeStruct((N_NODES, 1), jnp.float32),
    )(h, fc2[0], fc2[1].reshape(1, -1), fc3[0], fc3[1].reshape(1, -1))


def kernel(x, edge_index, edge_attr, params):
    src = edge_index[0]
    dst = edge_index[1]

    ones_rows = jnp.ones((N_EDGES, 16), jnp.float32)
    cnt2 = _sc_scatter_add(ones_rows, dst, 16)
    h, inv = _fc1(x, params['fc1'][0], params['fc1'][1], cnt2)

    for i, c in enumerate(params['convs']):
        hsrc = _sc_gather(h, src)
        msg = _edge_msg(edge_attr, hsrc, c['kernel'])
        s2 = _sc_scatter_add(msg, dst, D_NODE)
        h = _update(s2, inv, h, c['root'], c['bias'], do_gelu=(i < 3))

    return _head(h, params['fc2'], params['fc3'])


# speedup vs baseline: 3.6148x; 1.0009x over previous
import functools

import jax
import jax.numpy as jnp
from jax import lax
from jax.experimental import pallas as pl
from jax.experimental.pallas import tpu as pltpu
from jax.experimental.pallas import tpu_sc as plsc

N_NODES = 10000
N_EDGES = 160000
D_NODE = 32
E_BLK = 1280
CHUNK = 1000
N_CHUNKS = N_EDGES // CHUNK
NC = 2
NS = 16
NW = NC * NS
ROWS_PER_SUB = 624
ROWS_REM = N_NODES - NS * ROWS_PER_SUB


def _gelu(v):
    return 0.5 * v * (1.0 + lax.erf(v * (2.0 ** -0.5)))


def _sc_gather_body(h_hbm, src_hbm, out_hbm,
                    idx0, idx1, rows0, rows1, rows2,
                    g0, g1, o0, o1, o2):
    cid = lax.axis_index("c")
    sid = lax.axis_index("s")
    wid = sid * NC + cid
    per_w = N_CHUNKS // NW
    base = wid * per_w
    idx = [idx0, idx1]
    rows = [rows0, rows1, rows2]
    gsem = [g0, g1]
    osem = [o0, o1, o2]

    def load_idx(j):
        pltpu.sync_copy(src_hbm.at[pl.ds((base + j) * CHUNK, CHUNK)],
                        idx[j % 2])

    def fire_gather(j):
        pltpu.async_copy(h_hbm.at[idx[j % 2]], rows[j % 3], gsem[j % 2])

    load_idx(0)
    fire_gather(0)
    load_idx(1)
    fire_gather(1)
    for j in range(per_w):
        pltpu.make_async_copy(h_hbm.at[idx[j % 2]], rows[j % 3],
                              gsem[j % 2]).wait()
        pltpu.async_copy(rows[j % 3],
                         out_hbm.at[pl.ds((base + j) * CHUNK, CHUNK)],
                         osem[j % 3])
        nxt = j + 2
        if nxt < per_w:
            load_idx(nxt)
            if nxt >= 3:
                pltpu.make_async_copy(
                    rows[nxt % 3],
                    out_hbm.at[pl.ds((base + nxt - 3) * CHUNK, CHUNK)],
                    osem[nxt % 3]).wait()
            fire_gather(nxt)
    for j in range(max(0, per_w - 3), per_w):
        pltpu.make_async_copy(rows[j % 3],
                              out_hbm.at[pl.ds((base + j) * CHUNK, CHUNK)],
                              osem[j % 3]).wait()


def _sc_gather(h, src):
    mesh = plsc.VectorSubcoreMesh(core_axis_name="c", subcore_axis_name="s")
    return pl.kernel(
        _sc_gather_body,
        out_type=jax.ShapeDtypeStruct((N_EDGES, D_NODE), jnp.float32),
        mesh=mesh,
        compiler_params=pltpu.CompilerParams(use_tc_tiling_on_sc=False),
        scratch_types=[
            pltpu.VMEM((CHUNK,), jnp.int32),
            pltpu.VMEM((CHUNK,), jnp.int32),
            pltpu.VMEM((CHUNK, D_NODE), jnp.float32),
            pltpu.VMEM((CHUNK, D_NODE), jnp.float32),
            pltpu.VMEM((CHUNK, D_NODE), jnp.float32),
            pltpu.SemaphoreType.DMA,
            pltpu.SemaphoreType.DMA,
            pltpu.SemaphoreType.DMA,
            pltpu.SemaphoreType.DMA,
            pltpu.SemaphoreType.DMA,
        ],
    )(h, src)


def _sc_scatter_body(width, msg_hbm, dst_hbm, zeros_hbm, out_hbm,
                     idx0, idx1, rows0, rows1, acc,
                     i0, i1, m0, m1, ssem):
    cid = lax.axis_index("c")
    sid = lax.axis_index("s")
    pltpu.sync_copy(zeros_hbm.at[pl.ds(sid * ROWS_PER_SUB, ROWS_PER_SUB)],
                    acc.at[pl.ds(sid * ROWS_PER_SUB, ROWS_PER_SUB)])

    @pl.when(sid == 0)
    def _():
        pltpu.sync_copy(zeros_hbm.at[pl.ds(NS * ROWS_PER_SUB, ROWS_REM)],
                        acc.at[pl.ds(NS * ROWS_PER_SUB, ROWS_REM)])

    plsc.subcore_barrier()

    per_core = N_CHUNKS // NC
    per_w = per_core // NS
    base = cid * per_core + sid * per_w
    idx = [idx0, idx1]
    rows = [rows0, rows1]
    isem = [i0, i1]
    msem = [m0, m1]

    def load(j):
        b = j % 2
        pltpu.async_copy(dst_hbm.at[pl.ds((base + j) * CHUNK, CHUNK)],
                         idx[b], isem[b])
        pltpu.async_copy(msg_hbm.at[pl.ds((base + j) * CHUNK, CHUNK)],
                         rows[b], msem[b])

    load(0)
    for j in range(per_w):
        b = j % 2
        pltpu.make_async_copy(dst_hbm.at[pl.ds((base + j) * CHUNK, CHUNK)],
                              idx[b], isem[b]).wait()
        pltpu.make_async_copy(msg_hbm.at[pl.ds((base + j) * CHUNK, CHUNK)],
                              rows[b], msem[b]).wait()
        if j + 1 < per_w:
            load(j + 1)
        pltpu.async_copy(rows[b], acc.at[idx[b]], ssem, add=True)
        pltpu.make_async_copy(rows[b], acc.at[idx[b]], ssem).wait()
    plsc.subcore_barrier()
    pltpu.sync_copy(acc.at[pl.ds(sid * ROWS_PER_SUB, ROWS_PER_SUB)],
                    out_hbm.at[cid, pl.ds(sid * ROWS_PER_SUB, ROWS_PER_SUB)])

    @pl.when(sid == 0)
    def _():
        pltpu.sync_copy(acc.at[pl.ds(NS * ROWS_PER_SUB, ROWS_REM)],
                        out_hbm.at[cid, pl.ds(NS * ROWS_PER_SUB, ROWS_REM)])


def _sc_scatter_add(msg, dst, width):
    mesh = plsc.VectorSubcoreMesh(core_axis_name="c", subcore_axis_name="s")
    zeros = jnp.zeros((N_NODES, width), jnp.float32)
    return pl.kernel(
        functools.partial(_sc_scatter_body, width),
        out_type=jax.ShapeDtypeStruct((NC, N_NODES, width), jnp.float32),
        mesh=mesh,
        compiler_params=pltpu.CompilerParams(use_tc_tiling_on_sc=False),
        scratch_types=[
            pltpu.VMEM((CHUNK,), jnp.int32),
            pltpu.VMEM((CHUNK,), jnp.int32),
            pltpu.VMEM((CHUNK, width), jnp.float32),
            pltpu.VMEM((CHUNK, width), jnp.float32),
            pltpu.VMEM_SHARED((N_NODES, width), jnp.float32),
            pltpu.SemaphoreType.DMA,
            pltpu.SemaphoreType.DMA,
            pltpu.SemaphoreType.DMA,
            pltpu.SemaphoreType.DMA,
            pltpu.SemaphoreType.DMA,
        ],
    )(msg, dst, zeros)


def _fc1_body(x_ref, w_ref, b_ref, cnt_ref, h_ref, inv_ref):
    h_ref[...] = jnp.dot(x_ref[...], w_ref[...],
                         preferred_element_type=jnp.float32) + b_ref[...]
    cnt = cnt_ref[0, :, 0:1] + cnt_ref[1, :, 0:1]
    inv_ref[...] = 1.0 / jnp.maximum(cnt, 1.0)


def _fc1(x, w, b, cnt2):
    return pl.pallas_call(
        _fc1_body,
        out_shape=(
            jax.ShapeDtypeStruct((N_NODES, D_NODE), jnp.float32),
            jax.ShapeDtypeStruct((N_NODES, 1), jnp.float32),
        ),
    )(x, w, b.reshape(1, -1), cnt2)


def _edge_body(ea_ref, hs_ref, w1_ref, b1_ref, w2_ref, b2_ref,
               w3_ref, b3_ref, out_ref):
    eaT = ea_ref[...].T
    e1 = jnp.maximum(
        jnp.dot(w1_ref[...], eaT,
                preferred_element_type=jnp.float32) + b1_ref[...], 0.0)
    e2 = jnp.maximum(
        jnp.dot(w2_ref[...], e1,
                preferred_element_type=jnp.float32) + b2_ref[...], 0.0)
    wt = jnp.dot(w3_ref[...].astype(jnp.bfloat16), e2.astype(jnp.bfloat16),
                 preferred_element_type=jnp.float32) + b3_ref[...]
    wt3 = wt.reshape(D_NODE, D_NODE, E_BLK)
    hs = hs_ref[...].T
    msgT = jnp.sum(wt3 * hs[:, None, :], axis=0)
    out_ref[...] = msgT.T


def _edge_msg(ea, hsrc, kparams):
    (w1, b1), (w2, b2), (w3, b3) = kparams
    grid = N_EDGES // E_BLK
    full = lambda shape: pl.BlockSpec(shape, lambda j: (0, 0))
    return pl.pallas_call(
        _edge_body,
        grid=(grid,),
        in_specs=[
            pl.BlockSpec((E_BLK, 16), lambda j: (j, 0)),
            pl.BlockSpec((E_BLK, D_NODE), lambda j: (j, 0)),
            full((64, 16)), full((64, 1)),
            full((128, 64)), full((128, 1)),
            full((1024, 128)), full((1024, 1)),
        ],
        out_specs=pl.BlockSpec((E_BLK, D_NODE), lambda j: (j, 0)),
        out_shape=jax.ShapeDtypeStruct((N_EDGES, D_NODE), jnp.float32),
    )(ea, hsrc, w1.T, b1.reshape(-1, 1), w2.T, b2.reshape(-1, 1),
      w3.T, b3.reshape(-1, 1))


def _update_body(do_gelu, s2_ref, inv_ref, h_ref, root_ref, bias_ref, out_ref):
    aggr = (s2_ref[0] + s2_ref[1]) * inv_ref[...]
    v = aggr + jnp.dot(h_ref[...], root_ref[...],
                       preferred_element_type=jnp.float32) + bias_ref[...]
    if do_gelu:
        v = _gelu(v)
    out_ref[...] = v


def _update(s2, inv, h, root, bias, do_gelu):
    return pl.pallas_call(
        functools.partial(_update_body, do_gelu),
        out_shape=jax.ShapeDtypeStruct((N_NODES, D_NODE), jnp.float32),
    )(s2, inv, h, root, bias.reshape(1, -1))


def _head_body(h_ref, w2_ref, b2_ref, w3_ref, b3_ref, out_ref):
    v = jnp.dot(h_ref[...], w2_ref[...],
                preferred_element_type=jnp.float32) + b2_ref[...]
    v = _gelu(v)
    out_ref[...] = jnp.dot(v, w3_ref[...],
                           preferred_element_type=jnp.float32) + b3_ref[...]


def _head(h, fc2, fc3):
    return pl.pallas_call(
        _head_body,
        out_shape=jax.ShapeDtypeStruct((N_NODES, 1), jnp.float32),
    )(h, fc2[0], fc2[1].reshape(1, -1), fc3[0], fc3[1].reshape(1, -1))


def kernel(x, edge_index, edge_attr, params):
    src = edge_index[0]
    dst = edge_index[1]

    ones_rows = jnp.ones((N_EDGES, 16), jnp.float32)
    cnt2 = _sc_scatter_add(ones_rows, dst, 16)
    h, inv = _fc1(x, params['fc1'][0], params['fc1'][1], cnt2)

    for i, c in enumerate(params['convs']):
        hsrc = _sc_gather(h, src)
        msg = _edge_msg(edge_attr, hsrc, c['kernel'])
        s2 = _sc_scatter_add(msg, dst, D_NODE)
        h = _update(s2, inv, h, c['root'], c['bias'], do_gelu=(i < 3))

    return _head(h, params['fc2'], params['fc3'])
